# baseline (speedup 1.0000x reference)
import functools
import jax
import jax.numpy as jnp
from jax import lax
from jax.experimental import pallas as pl
from jax.experimental.pallas import tpu as pltpu
from jax.experimental.pallas import tpu_sc as plsc

N = 10000
D = 256
H = 128
E = 160000
NC = 2
NS = 16
CHUNK = 128
NCH = 84
E_PAD = NS * NCH * CHUNK
NACC = 10112
ZR = NACC // NS
WB = ZR
NBUF = 2
ROWB = 1024


@functools.cache
def _mesh():
    return plsc.VectorSubcoreMesh(
        core_axis_name="c", subcore_axis_name="s",
        num_cores=NC, num_subcores=NS)


NW = NC * NS
E_PAD_DEG = 160256
EPW = E_PAD_DEG // NW


def _deg_body(dstp_w, out, didx, hist):
    c = lax.axis_index("c")
    s = lax.axis_index("s")
    w = c * NS + s
    zv = jnp.zeros((16,), jnp.float32)

    def zero(i, carry):
        hist[pl.ds(i * 16, 16)] = zv
        return carry

    lax.fori_loop(0, NACC // 16, zero, 0)
    pltpu.sync_copy(dstp_w.at[w], didx)
    ones = jnp.ones((16,), jnp.float32)

    def step(g, carry):
        iv = didx[pl.ds(g * 16, 16)]
        plsc.addupdate_scatter(hist, [iv], ones)
        return carry

    lax.fori_loop(0, EPW // 16, step, 0)
    pltpu.sync_copy(hist, out.at[w])


@functools.cache
def _deg_call():
    return functools.partial(
        pl.kernel,
        out_type=jax.ShapeDtypeStruct((NW, NACC), jnp.float32),
        mesh=_mesh(),
        compiler_params=pltpu.CompilerParams(needs_layout_passes=False),
        scratch_types=[
            pltpu.VMEM((EPW,), jnp.int32),
            pltpu.VMEM((NACC,), jnp.float32),
        ],
    )(_deg_body)


def _scat_body(ys, srcp2, dstp_s, out, sb0, sb1, sb2, db0, db1, db2, db3,
               rows0, rows1, rows2, acc, i0, i1, i2, j0, j1, j2, j3,
               g0, g1, g2, s0, s1, s2):
    c = lax.axis_index("c")
    s = lax.axis_index("s")
    w = c * NS + s
    rows = (rows0, rows1, rows2)
    sb = (sb0, sb1, sb2)
    db = (db0, db1, db2, db3)
    isem = (i0, i1, i2)
    jsem = (j0, j1, j2, j3)
    gsem = (g0, g1, g2)
    ssem = (s0, s1, s2)

    zv = jnp.zeros((16,), jnp.float32)

    def zrow(i, carry):
        for j in range(H // 16):
            rows0[i, pl.ds(j * 16, 16)] = zv
        return carry

    lax.fori_loop(0, CHUNK, zrow, 0)
    nfull = ZR // CHUNK
    rem = ZR - nfull * CHUNK
    for j in range(nfull):
        pltpu.sync_copy(rows0, acc.at[pl.ds(s * ZR + j * CHUNK, CHUNK)])
    pltpu.sync_copy(rows0.at[pl.ds(0, rem)],
                    acc.at[pl.ds(s * ZR + nfull * CHUNK, rem)])
    plsc.subcore_barrier()

    hdummy = ys.at[pl.ds(0, CHUNK)]
    idummy = srcp2.at[w, 0]
    for b in range(3):
        pltpu.async_copy(srcp2.at[w, b], sb[b], isem[b])
    pltpu.async_copy(dstp_s.at[s, 0], db0, j0)
    for b in range(2):
        pltpu.make_async_copy(idummy, sb[b], isem[b]).wait()
        pltpu.async_copy(ys.at[sb[b]], rows[b], gsem[b])
    pltpu.async_copy(hdummy, rows2, s2)

    def body12(q, carry):
        k0 = q * 12
        for t in range(12):
            k = k0 + t
            j = t % 3
            r = t % 4
            b2 = (j + 2) % 3
            pltpu.make_async_copy(hdummy, rows[j], gsem[j]).wait()
            pltpu.make_async_copy(idummy, db[r], jsem[r]).wait()
            pltpu.async_copy(rows[j], acc.at[db[r]], ssem[j], add=True)
            nk = jnp.where(k + 2 < NCH, k + 2, 0)
            pltpu.make_async_copy(idummy, sb[b2], isem[b2]).wait()
            pltpu.make_async_copy(hdummy, rows[b2], ssem[b2]).wait()
            pltpu.async_copy(ys.at[sb[b2]], rows[b2], gsem[b2])
            nk3 = jnp.where(k + 3 < NCH, k + 3, 0)
            pltpu.async_copy(srcp2.at[w, nk3], sb[j], isem[j])
            nkd = jnp.where(k + 1 < NCH, k + 1, 0)
            pltpu.async_copy(dstp_s.at[s, nkd], db[(r + 1) % 4],
                             jsem[(r + 1) % 4])
        return carry

    lax.fori_loop(0, NCH // 12, body12, 0)
    for b in range(2):
        pltpu.make_async_copy(hdummy, rows[b], gsem[b]).wait()
    pltpu.make_async_copy(hdummy, rows2, s2).wait()
    pltpu.make_async_copy(idummy, sb2, i2).wait()
    pltpu.make_async_copy(idummy, db0, j0).wait()
    plsc.subcore_barrier()
    pltpu.sync_copy(acc.at[pl.ds(s * WB, WB)],
                    out.at[pl.ds(c * NACC + s * WB, WB)])


@functools.cache
def _scat_call():
    return functools.partial(
        pl.kernel,
        out_type=jax.ShapeDtypeStruct((NC * NACC, H), jnp.float32),
        mesh=_mesh(),
        scratch_types=[
            pltpu.VMEM((CHUNK,), jnp.int32),
            pltpu.VMEM((CHUNK,), jnp.int32),
            pltpu.VMEM((CHUNK,), jnp.int32),
            pltpu.VMEM((CHUNK,), jnp.int32),
            pltpu.VMEM((CHUNK,), jnp.int32),
            pltpu.VMEM((CHUNK,), jnp.int32),
            pltpu.VMEM((CHUNK,), jnp.int32),
            pltpu.VMEM((CHUNK, H), jnp.float32),
            pltpu.VMEM((CHUNK, H), jnp.float32),
            pltpu.VMEM((CHUNK, H), jnp.float32),
            pltpu.VMEM_SHARED((NACC, H), jnp.float32),
            pltpu.SemaphoreType.DMA,
            pltpu.SemaphoreType.DMA,
            pltpu.SemaphoreType.DMA,
            pltpu.SemaphoreType.DMA,
            pltpu.SemaphoreType.DMA,
            pltpu.SemaphoreType.DMA,
            pltpu.SemaphoreType.DMA,
            pltpu.SemaphoreType.DMA,
            pltpu.SemaphoreType.DMA,
            pltpu.SemaphoreType.DMA,
            pltpu.SemaphoreType.DMA,
            pltpu.SemaphoreType.DMA,
            pltpu.SemaphoreType.DMA,
        ],
    )(_scat_body)


def _dinv(p_ref):
    colsum = lax.dot_general(p_ref[...], jnp.ones((NW, 1), jnp.float32),
                             (((0,), (0,)), ((), ())),
                             preferred_element_type=jnp.float32)
    return lax.rsqrt(colsum + 1.0)


def _mm_body(x_ref, w_ref, p0_ref, y2_ref):
    dinv = _dinv(p0_ref)
    y = jnp.dot(x_ref[...], w_ref[...], preferred_element_type=jnp.float32)
    y = y * dinv
    y2_ref[0] = y[:, :H]
    y2_ref[1] = y[:, H:]


def _mm(x, w, p0):
    grid = (pl.cdiv(N, ROWB),)
    return pl.pallas_call(
        _mm_body,
        grid=grid,
        in_specs=[
            pl.BlockSpec((ROWB, D), lambda i: (i, 0)),
            pl.BlockSpec((D, D), lambda i: (0, 0)),
            pl.BlockSpec((NW, ROWB), lambda i: (0, i)),
        ],
        out_specs=pl.BlockSpec((NC, ROWB, H), lambda i: (0, i, 0)),
        out_shape=jax.ShapeDtypeStruct((NC, N, H), jnp.float32),
    )(x, w, p0)


def _comb_body(a0_ref, a1_ref, y2_ref, p0_ref, b_ref, o_ref):
    dinv = _dinv(p0_ref)
    z0 = dinv * (a0_ref[...] + y2_ref[0]) + b_ref[:, :H]
    z1 = dinv * (a1_ref[...] + y2_ref[1]) + b_ref[:, H:]
    o_ref[:, :H] = jnp.maximum(z0, 0.0)
    o_ref[:, H:] = jnp.maximum(z1, 0.0)


def _comb(a0, a1, y2, p0, b2d):
    grid = (pl.cdiv(N, ROWB),)
    return pl.pallas_call(
        _comb_body,
        grid=grid,
        in_specs=[
            pl.BlockSpec((ROWB, H), lambda i: (i, 0)),
            pl.BlockSpec((ROWB, H), lambda i: (i, 0)),
            pl.BlockSpec((NC, ROWB, H), lambda i: (0, i, 0)),
            pl.BlockSpec((NW, ROWB), lambda i: (0, i)),
            pl.BlockSpec((1, D), lambda i: (0, 0)),
        ],
        out_specs=pl.BlockSpec((ROWB, D), lambda i: (i, 0)),
        out_shape=jax.ShapeDtypeStruct((N, D), jnp.float32),
    )(a0, a1, y2, p0, b2d)


def kernel(edge_index, edge_label_index, W0, b0, W1, b1, W2, b2):
    x = edge_index
    src = edge_label_index[0]
    dst = edge_label_index[1]
    pad = E_PAD - E
    srcp = jnp.concatenate([src, jnp.zeros((pad,), jnp.int32)])
    dstp = jnp.concatenate([dst, jnp.full((pad,), N, jnp.int32)])
    srcp2 = jnp.stack([srcp, srcp + N]).reshape(NC * NS, NCH, CHUNK)
    dstp_s = dstp.reshape(NS, NCH, CHUNK)

    dstp_deg = jnp.concatenate(
        [dst, jnp.full((E_PAD_DEG - E,), N, jnp.int32)])
    p0 = _deg_call()(dstp_deg.reshape(NW, EPW))

    for W, b in ((W0, b0), (W1, b1), (W2, b2)):
        y2 = _mm(x, W, p0)
        sc = _scat_call()(y2.reshape(NC * N, H), srcp2, dstp_s)
        x = _comb(sc[:N], sc[NACC:NACC + N], y2, p0, b.reshape(1, D))
    return x


# speedup vs baseline: 2.3922x; 2.3922x over previous
import functools
import jax
import jax.numpy as jnp
from jax import lax
from jax.experimental import pallas as pl
from jax.experimental.pallas import tpu as pltpu
from jax.experimental.pallas import tpu_sc as plsc

N = 10000
D = 256
H = 128
E = 160000
NC = 2
NS = 16
CHUNK = 128
NCH = 80
NCHD = 40
E_PAD = NS * NCH * CHUNK
NACC = 10112
ZR = NACC // NS
WB = ZR
NBUF = 2
ROWB = 1024


@functools.cache
def _mesh():
    return plsc.VectorSubcoreMesh(
        core_axis_name="c", subcore_axis_name="s",
        num_cores=NC, num_subcores=NS)


NW = NC * NS
EPW = E_PAD // NW


def _deg_body(dstp_w, out, didx, hist):
    c = lax.axis_index("c")
    s = lax.axis_index("s")
    w = c * NS + s
    zv = jnp.zeros((16,), jnp.float32)

    def zero(i, carry):
        hist[pl.ds(i * 16, 16)] = zv
        return carry

    lax.fori_loop(0, NACC // 16, zero, 0)
    pltpu.sync_copy(dstp_w.at[w], didx)
    ones = jnp.ones((16,), jnp.float32)

    def step(g, carry):
        iv = didx[pl.ds(g * 16, 16)]
        plsc.addupdate_scatter(hist, [iv], ones)
        return carry

    lax.fori_loop(0, EPW // 16, step, 0)
    pltpu.sync_copy(hist, out.at[w])


@functools.cache
def _deg_call():
    return functools.partial(
        pl.kernel,
        out_type=jax.ShapeDtypeStruct((NW, NACC), jnp.float32),
        mesh=_mesh(),
        compiler_params=pltpu.CompilerParams(needs_layout_passes=False),
        scratch_types=[
            pltpu.VMEM((EPW,), jnp.int32),
            pltpu.VMEM((NACC,), jnp.float32),
        ],
    )(_deg_body)


def _scat_body(ys, srcp2, dstp_s, out, sb0, sb1, didx,
               rows0, rows1, acc, i0, i1, g0, g1, s0, s1):
    c = lax.axis_index("c")
    s = lax.axis_index("s")
    w = c * NS + s
    rows = (rows0, rows1)
    sb = (sb0, sb1)
    isem = (i0, i1)
    gsem = (g0, g1)
    ssem = (s0, s1)

    zv = jnp.zeros((16,), jnp.float32)

    def zrow(i, carry):
        for j in range(H // 16):
            rows0[i, pl.ds(j * 16, 16)] = zv
        return carry

    lax.fori_loop(0, CHUNK, zrow, 0)
    for j in range(4):
        pltpu.sync_copy(rows0, acc.at[pl.ds(s * ZR + j * CHUNK, CHUNK)])
    pltpu.sync_copy(rows0.at[pl.ds(0, ZR - 4 * CHUNK)],
                    acc.at[pl.ds(s * ZR + 4 * CHUNK, ZR - 4 * CHUNK)])

    pltpu.sync_copy(dstp_s.at[s], didx)
    plsc.subcore_barrier()

    for b in range(2):
        pltpu.async_copy(srcp2.at[w, b], sb[b], isem[b])

    hdummy = ys.at[pl.ds(0, CHUNK)]

    def pair(q, carry):
        k0 = q * 2
        for b in range(2):
            k = k0 + b
            pltpu.make_async_copy(srcp2.at[w, 0], sb[b], isem[b]).wait()
            pltpu.make_async_copy(hdummy, rows[b], ssem[b]).wait()
            pltpu.async_copy(ys.at[sb[b]], rows[b], gsem[b])
            pltpu.make_async_copy(hdummy, rows[b], gsem[b]).wait()
            pltpu.async_copy(rows[b], acc.at[didx.at[k]], ssem[b], add=True)
            nk = k + 2
            nk = jnp.where(nk < NCH, nk, 0)
            pltpu.async_copy(srcp2.at[w, nk], sb[b], isem[b])
        return carry

    for b in range(2):
        pltpu.async_copy(hdummy, rows[b], ssem[b])
    lax.fori_loop(0, NCH // 2, pair, 0)
    for b in range(2):
        pltpu.make_async_copy(hdummy, rows[b], ssem[b]).wait()
        pltpu.make_async_copy(srcp2.at[w, 0], sb[b], isem[b]).wait()
    plsc.subcore_barrier()
    pltpu.sync_copy(acc.at[pl.ds(s * WB, WB)],
                    out.at[pl.ds(c * NACC + s * WB, WB)])


@functools.cache
def _scat_call():
    return functools.partial(
        pl.kernel,
        out_type=jax.ShapeDtypeStruct((NC * NACC, H), jnp.float32),
        mesh=_mesh(),
        scratch_types=[
            pltpu.VMEM((CHUNK,), jnp.int32),
            pltpu.VMEM((CHUNK,), jnp.int32),
            pltpu.VMEM((NCH, CHUNK), jnp.int32),
            pltpu.VMEM((CHUNK, H), jnp.float32),
            pltpu.VMEM((CHUNK, H), jnp.float32),
            pltpu.VMEM_SHARED((NACC, H), jnp.float32),
            pltpu.SemaphoreType.DMA,
            pltpu.SemaphoreType.DMA,
            pltpu.SemaphoreType.DMA,
            pltpu.SemaphoreType.DMA,
            pltpu.SemaphoreType.DMA,
            pltpu.SemaphoreType.DMA,
        ],
    )(_scat_body)


def _dinv(p_ref):
    colsum = lax.dot_general(p_ref[...], jnp.ones((NW, 1), jnp.float32),
                             (((0,), (0,)), ((), ())),
                             preferred_element_type=jnp.float32)
    return lax.rsqrt(colsum + 1.0)


def _mm_body(x_ref, w_ref, p0_ref, y2_ref):
    dinv = _dinv(p0_ref)
    y = jnp.dot(x_ref[...], w_ref[...], preferred_element_type=jnp.float32)
    y = y * dinv
    y2_ref[0] = y[:, :H]
    y2_ref[1] = y[:, H:]


def _mm(x, w, p0):
    grid = (pl.cdiv(N, ROWB),)
    return pl.pallas_call(
        _mm_body,
        grid=grid,
        in_specs=[
            pl.BlockSpec((ROWB, D), lambda i: (i, 0)),
            pl.BlockSpec((D, D), lambda i: (0, 0)),
            pl.BlockSpec((NW, ROWB), lambda i: (0, i)),
        ],
        out_specs=pl.BlockSpec((NC, ROWB, H), lambda i: (0, i, 0)),
        out_shape=jax.ShapeDtypeStruct((NC, N, H), jnp.float32),
    )(x, w, p0)


def _comb_body(a0_ref, a1_ref, y2_ref, p0_ref, b_ref, o_ref):
    dinv = _dinv(p0_ref)
    z0 = dinv * (a0_ref[...] + y2_ref[0]) + b_ref[:, :H]
    z1 = dinv * (a1_ref[...] + y2_ref[1]) + b_ref[:, H:]
    o_ref[:, :H] = jnp.maximum(z0, 0.0)
    o_ref[:, H:] = jnp.maximum(z1, 0.0)


def _comb(a0, a1, y2, p0, b2d):
    grid = (pl.cdiv(N, ROWB),)
    return pl.pallas_call(
        _comb_body,
        grid=grid,
        in_specs=[
            pl.BlockSpec((ROWB, H), lambda i: (i, 0)),
            pl.BlockSpec((ROWB, H), lambda i: (i, 0)),
            pl.BlockSpec((NC, ROWB, H), lambda i: (0, i, 0)),
            pl.BlockSpec((NW, ROWB), lambda i: (0, i)),
            pl.BlockSpec((1, D), lambda i: (0, 0)),
        ],
        out_specs=pl.BlockSpec((ROWB, D), lambda i: (i, 0)),
        out_shape=jax.ShapeDtypeStruct((N, D), jnp.float32),
    )(a0, a1, y2, p0, b2d)


def _fuse_body(a0_ref, a1_ref, y2_ref, p0_ref, b_ref, w_ref, y2o_ref):
    dinv = _dinv(p0_ref)
    z0 = dinv * (a0_ref[...] + y2_ref[0]) + b_ref[:, :H]
    z1 = dinv * (a1_ref[...] + y2_ref[1]) + b_ref[:, H:]
    z = jnp.concatenate([jnp.maximum(z0, 0.0), jnp.maximum(z1, 0.0)], axis=1)
    y = jnp.dot(z, w_ref[...], preferred_element_type=jnp.float32) * dinv
    y2o_ref[0] = y[:, :H]
    y2o_ref[1] = y[:, H:]


def _fuse(a0, a1, y2, p0, b2d, w):
    grid = (pl.cdiv(N, ROWB),)
    return pl.pallas_call(
        _fuse_body,
        grid=grid,
        in_specs=[
            pl.BlockSpec((ROWB, H), lambda i: (i, 0)),
            pl.BlockSpec((ROWB, H), lambda i: (i, 0)),
            pl.BlockSpec((NC, ROWB, H), lambda i: (0, i, 0)),
            pl.BlockSpec((NW, ROWB), lambda i: (0, i)),
            pl.BlockSpec((1, D), lambda i: (0, 0)),
            pl.BlockSpec((D, D), lambda i: (0, 0)),
        ],
        out_specs=pl.BlockSpec((NC, ROWB, H), lambda i: (0, i, 0)),
        out_shape=jax.ShapeDtypeStruct((NC, N, H), jnp.float32),
    )(a0, a1, y2, p0, b2d, w)


def kernel(edge_index, edge_label_index, W0, b0, W1, b1, W2, b2):
    x = edge_index
    src = edge_label_index[0]
    dst = edge_label_index[1]
    pad = E_PAD - E
    srcp = jnp.concatenate([src, jnp.zeros((pad,), jnp.int32)])
    dstp = jnp.concatenate([dst, jnp.full((pad,), N, jnp.int32)])
    srcp2 = jnp.stack([srcp, srcp + N]).reshape(NC * NS, NCH, CHUNK)
    dstp_s = dstp.reshape(NS, NCH, CHUNK)

    p0 = _deg_call()(dstp.reshape(NW, EPW))

    y2 = _mm(x, W0, p0)
    for b, Wn in ((b0, W1), (b1, W2)):
        sc = _scat_call()(y2.reshape(NC * N, H), srcp2, dstp_s)
        y2 = _fuse(sc[:N], sc[NACC:NACC + N], y2, p0, b.reshape(1, D), Wn)
    sc = _scat_call()(y2.reshape(NC * N, H), srcp2, dstp_s)
    return _comb(sc[:N], sc[NACC:NACC + N], y2, p0, b2.reshape(1, D))
